# Initial kernel scaffold; baseline (speedup 1.0000x reference)
import jax
import jax.numpy as jnp
from jax.experimental import pallas as pl


def kernel(x, weight):
    raise NotImplementedError("write your pallas kernel here")


# speedup vs baseline: 1.8161x; 1.8161x over previous
import functools

import jax
import jax.numpy as jnp
from jax import lax
from jax.experimental import pallas as pl
from jax.experimental.pallas import tpu as pltpu
from jax.experimental.pallas import tpu_sc as plsc

VOCAB = 1000000
EMBED_DIM = 64
BATCH = 16384
HIST = 50
B = BATCH * HIST

NC, NS = 2, 16
NW = NC * NS
G = 128
GPW = B // (G * NW)
K = 5
CH = K * G
NCHUNK = GPW // K

_mesh = plsc.VectorSubcoreMesh(core_axis_name="c", subcore_axis_name="s")


@functools.partial(
    pl.kernel,
    mesh=_mesh,
    out_type=jax.ShapeDtypeStruct((B, EMBED_DIM), jnp.float32),
    scratch_types=[
        pltpu.VMEM((CH,), jnp.int32),
        pltpu.VMEM((CH, EMBED_DIM), jnp.float32),
        pltpu.SemaphoreType.DMA,
    ],
    compiler_params=pltpu.CompilerParams(use_tc_tiling_on_sc=False),
)
def _gather(table, idx, out, idx_v, rows_v, sem):
    wid = lax.axis_index("s") * NC + lax.axis_index("c")
    base_r = wid * GPW * G

    def chunk(g, carry):
        r0 = base_r + g * CH
        pltpu.sync_copy(idx.at[pl.ds(r0, CH)], idx_v)
        copies = []
        for j in range(K):
            copies.append(
                pltpu.async_copy(
                    table.at[idx_v.at[pl.ds(j * G, G)]],
                    rows_v.at[pl.ds(j * G, G)],
                    sem,
                )
            )
        for c in copies:
            c.wait()
        pltpu.sync_copy(rows_v, out.at[pl.ds(r0, CH)])
        return carry

    lax.fori_loop(0, NCHUNK, chunk, 0)


def kernel(x, weight):
    xf = x.reshape(B)
    out = _gather(weight, xf)
    return out.reshape(BATCH, HIST, EMBED_DIM)


# speedup vs baseline: 1.8743x; 1.0321x over previous
import functools

import jax
import jax.numpy as jnp
from jax import lax
from jax.experimental import pallas as pl
from jax.experimental.pallas import tpu as pltpu
from jax.experimental.pallas import tpu_sc as plsc

VOCAB = 1000000
EMBED_DIM = 64
BATCH = 16384
HIST = 50
B = BATCH * HIST

NC, NS = 2, 16
NW = NC * NS
G = 128
GPW = B // (G * NW)
K = 5
CH = K * G
NCHUNK = GPW // K

_mesh = plsc.VectorSubcoreMesh(core_axis_name="c", subcore_axis_name="s")


@functools.partial(
    pl.kernel,
    mesh=_mesh,
    out_type=jax.ShapeDtypeStruct((B, EMBED_DIM), jnp.float32),
    scratch_types=[
        pltpu.VMEM((2, CH), jnp.int32),
        pltpu.VMEM((2, CH, EMBED_DIM), jnp.float32),
        pltpu.SemaphoreType.DMA,
        pltpu.SemaphoreType.DMA,
        pltpu.SemaphoreType.DMA,
        pltpu.SemaphoreType.DMA,
        pltpu.SemaphoreType.DMA,
        pltpu.SemaphoreType.DMA,
    ],
    compiler_params=pltpu.CompilerParams(use_tc_tiling_on_sc=False),
)
def _gather(table, idx, out, idx_v, rows_v, si0, si1, sg0, sg1, so0, so1):
    wid = lax.axis_index("s") * NC + lax.axis_index("c")
    base_r = wid * GPW * G
    si = (si0, si1)
    sg = (sg0, sg1)
    so = (so0, so1)

    def idx_copy(g, b):
        return pltpu.make_async_copy(
            idx.at[pl.ds(base_r + g * CH, CH)], idx_v.at[b], si[b]
        )

    def out_copy(g, b):
        return pltpu.make_async_copy(
            rows_v.at[b], out.at[pl.ds(base_r + g * CH, CH)], so[b]
        )

    def run_chunk(g, b, *, wait_out, prefetch):
        if wait_out:
            out_copy(g - 2, b).wait()
        idx_copy(g, b).wait()
        gathers = [
            pltpu.async_copy(
                table.at[idx_v.at[b].at[pl.ds(j * G, G)]],
                rows_v.at[b].at[pl.ds(j * G, G)],
                sg[b],
            )
            for j in range(K)
        ]
        for c in gathers:
            c.wait()
        if prefetch:
            idx_copy(g + 2, b).start()
        out_copy(g, b).start()

    idx_copy(0, 0).start()
    idx_copy(1, 1).start()
    run_chunk(0, 0, wait_out=False, prefetch=True)
    run_chunk(1, 1, wait_out=False, prefetch=True)

    def pair(gg, carry):
        g = gg * 2
        run_chunk(g, 0, wait_out=True, prefetch=True)
        run_chunk(g + 1, 1, wait_out=True, prefetch=True)
        return carry

    lax.fori_loop(1, NCHUNK // 2 - 1, pair, 0)

    run_chunk(NCHUNK - 2, 0, wait_out=True, prefetch=False)
    run_chunk(NCHUNK - 1, 1, wait_out=True, prefetch=False)
    out_copy(NCHUNK - 2, 0).wait()
    out_copy(NCHUNK - 1, 1).wait()


def kernel(x, weight):
    xf = x.reshape(B)
    out = _gather(weight, xf)
    return out.reshape(BATCH, HIST, EMBED_DIM)


# speedup vs baseline: 1.8763x; 1.0010x over previous
import functools

import jax
import jax.numpy as jnp
from jax import lax
from jax.experimental import pallas as pl
from jax.experimental.pallas import tpu as pltpu
from jax.experimental.pallas import tpu_sc as plsc

VOCAB = 1000000
EMBED_DIM = 64
BATCH = 16384
HIST = 50
B = BATCH * HIST

NC, NS = 2, 16
NW = NC * NS
G = 128
GPW = B // (G * NW)
K = 5
CH = K * G
NCHUNK = GPW // K
NI = 4

_mesh = plsc.VectorSubcoreMesh(core_axis_name="c", subcore_axis_name="s")


@functools.partial(
    pl.kernel,
    mesh=_mesh,
    out_type=jax.ShapeDtypeStruct((B, EMBED_DIM), jnp.float32),
    scratch_types=[
        pltpu.VMEM((NI, CH), jnp.int32),
        pltpu.VMEM((2, CH, EMBED_DIM), jnp.float32),
        pltpu.SemaphoreType.DMA,
        pltpu.SemaphoreType.DMA,
        pltpu.SemaphoreType.DMA,
        pltpu.SemaphoreType.DMA,
        pltpu.SemaphoreType.DMA,
        pltpu.SemaphoreType.DMA,
        pltpu.SemaphoreType.DMA,
        pltpu.SemaphoreType.DMA,
    ],
    compiler_params=pltpu.CompilerParams(use_tc_tiling_on_sc=False),
)
def _gather(table, idx, out, idx_v, rows_v,
            si0, si1, si2, si3, sg0, sg1, so0, so1):
    wid = lax.axis_index("s") * NC + lax.axis_index("c")
    base_r = wid * GPW * G
    si = (si0, si1, si2, si3)
    sg = (sg0, sg1)
    so = (so0, so1)

    def idx_copy(g, a):
        return pltpu.make_async_copy(
            idx.at[pl.ds(base_r + g * CH, CH)], idx_v.at[a], si[a]
        )

    def out_copy(g, b):
        return pltpu.make_async_copy(
            rows_v.at[b], out.at[pl.ds(base_r + g * CH, CH)], so[b]
        )

    def fire_gathers(g, a, b):
        for j in range(K):
            pltpu.async_copy(
                table.at[idx_v.at[a].at[pl.ds(j * G, G)]],
                rows_v.at[b].at[pl.ds(j * G, G)],
                sg[b],
            )

    def drain_gathers(b):
        for j in range(K):
            pltpu.make_async_copy(
                table.at[pl.ds(0, G)],
                rows_v.at[b].at[pl.ds(j * G, G)],
                sg[b],
            ).wait()

    def run_chunk(g, *, b, a, wait_out, drain_prev, prefetch):
        if wait_out:
            out_copy(g - 2, b).wait()
        idx_copy(g, a).wait()
        fire_gathers(g, a, b)
        if drain_prev:
            drain_gathers(b ^ 1)
            out_copy(g - 1, b ^ 1).start()
        if prefetch:
            idx_copy(g + 3, (g + 3) % NI).start()

    for a in range(NI):
        idx_copy(a, a).start()

    run_chunk(0, b=0, a=0, wait_out=False, drain_prev=False, prefetch=False)
    run_chunk(1, b=1, a=1, wait_out=False, drain_prev=True, prefetch=True)

    def pair_body(gg, carry):
        g = gg * 2

        def chunk_dyn(g, b):
            out_copy(g - 2, b).wait()
            a = lax.rem(g, NI)
            for slot in range(NI):
                @pl.when(a == slot)
                def _():
                    idx_copy(g, slot).wait()
                    fire_gathers(g, slot, b)
            drain_gathers(b ^ 1)
            out_copy(g - 1, b ^ 1).start()
            for slot in range(NI):
                @pl.when(lax.rem(g + 3, NI) == slot)
                def _():
                    idx_copy(g + 3, slot).start()

        chunk_dyn(g, 0)
        chunk_dyn(g + 1, 1)
        return carry

    lax.fori_loop(1, 18, pair_body, 0)

    run_chunk(36, b=0, a=0, wait_out=True, drain_prev=True, prefetch=True)
    run_chunk(37, b=1, a=1, wait_out=True, drain_prev=True, prefetch=False)
    run_chunk(38, b=0, a=2, wait_out=True, drain_prev=True, prefetch=False)
    run_chunk(39, b=1, a=3, wait_out=True, drain_prev=True, prefetch=False)
    drain_gathers(1)
    out_copy(39, 1).start()
    out_copy(38, 0).wait()
    out_copy(39, 1).wait()


def kernel(x, weight):
    xf = x.reshape(B)
    out = _gather(weight, xf)
    return out.reshape(BATCH, HIST, EMBED_DIM)
